# baseline (speedup 1.0000x reference)
import functools

import jax
import jax.numpy as jnp
from jax.experimental import pallas as pl
import jax.experimental.pallas.tpu as pltpu

B = 8192
D = 256
K = 8192
L = 4

B_TILE = 512
K_TILE = 4096
CHUNK = 32

BIG = 3.0e38


def _argmin_body(emb_ref, res_ref, e2_ref, x2_ref, idx_ref, gv_ref, gi_ref,
                 *, n_k):
    k = pl.program_id(0)
    b = pl.program_id(1)
    lane_sl = pl.ds(b * B_TILE, B_TILE)

    rs16 = res_ref[...].astype(jnp.bfloat16)
    x2r = x2_ref[...]
    sub_iota = jax.lax.broadcasted_iota(jnp.int32, (8, B_TILE), 0)
    base = k * K_TILE

    def chunk(c, carry):
        rm, ra = carry
        eblk16 = emb_ref[pl.ds(c * CHUNK, CHUNK), :].astype(jnp.bfloat16)
        e2c = e2_ref[pl.ds(c * CHUNK, CHUNK), :]
        dots = jax.lax.dot_general(
            eblk16, rs16, (((1,), (1,)), ((), ())),
            preferred_element_type=jnp.float32)
        scores = (x2r + e2c) - 2.0 * dots
        for g in range(CHUNK // 8):
            v = scores[g * 8:(g + 1) * 8, :]
            kidx = sub_iota + (base + c * CHUNK + g * 8)
            better = v < rm
            rm = jnp.minimum(rm, v)
            ra = jnp.where(better, kidx, ra)
        return rm, ra

    rm0 = jnp.full((8, B_TILE), BIG, jnp.float32)
    ra0 = jnp.zeros((8, B_TILE), jnp.int32)
    rm, ra = jax.lax.fori_loop(0, K_TILE // CHUNK, chunk, (rm0, ra0))

    wv = jnp.min(rm, axis=0, keepdims=True)
    cand = jnp.where(rm == wv, ra, jnp.int32(K))
    wi = jnp.min(cand, axis=0, keepdims=True)

    @pl.when(k == 0)
    def _():
        gv_ref[:, lane_sl] = wv
        gi_ref[:, lane_sl] = wi

    @pl.when(k > 0)
    def _():
        acc = gv_ref[:, lane_sl]
        av = acc.astype(jnp.bfloat16).astype(jnp.float32)
        upd = wv < av
        gv_ref[:, lane_sl] = jnp.where(upd, wv, av)
        gi_ref[:, lane_sl] = jnp.where(upd, wi, gi_ref[:, lane_sl])

    @pl.when(k == n_k - 1)
    def _():
        idx_ref[...] = gi_ref[:, lane_sl][None]


def _matmul_argmin(res, emb, e2, x2):
    n_b = B // B_TILE
    n_k = K // K_TILE
    idx3 = pl.pallas_call(
        functools.partial(_argmin_body, n_k=n_k),
        grid=(n_k, n_b),
        in_specs=[
            pl.BlockSpec((K_TILE, D), lambda k, b: (k, 0)),
            pl.BlockSpec((B_TILE, D), lambda k, b: (b, 0)),
            pl.BlockSpec((K_TILE, 1), lambda k, b: (k, 0)),
            pl.BlockSpec((1, B_TILE), lambda k, b: (0, b)),
        ],
        out_specs=pl.BlockSpec((1, 1, B_TILE), lambda k, b: (b, 0, 0)),
        out_shape=jax.ShapeDtypeStruct((n_b, 1, B_TILE), jnp.int32),
        scratch_shapes=[
            pltpu.VMEM((1, B), jnp.float32),
            pltpu.VMEM((1, B), jnp.int32),
        ],
    )(emb, res, e2, x2)
    return idx3.reshape(B)


def kernel(residual, codebooks):
    res = residual
    idxs = []
    for l in range(L):
        emb = codebooks[l]
        e2 = jnp.sum(emb ** 2, axis=1)[:, None]
        x2 = jnp.sum(res ** 2, axis=1)[None, :]
        idx = _matmul_argmin(res, emb, e2, x2)
        q = jnp.take(emb, idx, axis=0)
        res = res - q
        idxs.append(idx)
    codes = jnp.stack(idxs, axis=1)
    quantized_sum = residual - res
    return quantized_sum, codes


# speedup vs baseline: 1.7454x; 1.7454x over previous
import functools

import jax
import jax.numpy as jnp
from jax.experimental import pallas as pl
import jax.experimental.pallas.tpu as pltpu

B = 8192
D = 256
K = 8192
L = 4

B_TILE = 512
K_TILE = 4096
CHUNK = 64

BIG = 3.0e38


def _argmin_body(emb_ref, res_ref, e2_ref, x2_ref, idx_ref, gv_ref, gi_ref,
                 *, n_k):
    k = pl.program_id(0)
    b = pl.program_id(1)
    lane_sl = pl.ds(b * B_TILE, B_TILE)

    rs16 = res_ref[...].astype(jnp.bfloat16)
    x2r = x2_ref[...]
    sub_iota = jax.lax.broadcasted_iota(jnp.int32, (8, B_TILE), 0)
    base = k * K_TILE

    def dot_chunk(c):
        eblk16 = emb_ref[pl.ds(c * CHUNK, CHUNK), :].astype(jnp.bfloat16)
        return jax.lax.dot_general(
            eblk16, rs16, (((1,), (1,)), ((), ())),
            preferred_element_type=jnp.float32)

    def process(c, dots, rm, ra):
        e2c = e2_ref[pl.ds(c * CHUNK, CHUNK), :]
        scores = (x2r + e2c) - 2.0 * dots
        for g in range(CHUNK // 8):
            v = scores[g * 8:(g + 1) * 8, :]
            kidx = sub_iota + (base + c * CHUNK + g * 8)
            better = v < rm
            rm = jnp.minimum(rm, v)
            ra = jnp.where(better, kidx, ra)
        return rm, ra

    def chunk(c, carry):
        rm, ra, d_prev = carry
        d_cur = dot_chunk(c)
        rm, ra = process(c - 1, d_prev, rm, ra)
        return rm, ra, d_cur

    n_c = K_TILE // CHUNK
    rm0 = jnp.full((8, B_TILE), BIG, jnp.float32)
    ra0 = jnp.zeros((8, B_TILE), jnp.int32)
    rm, ra, d_last = jax.lax.fori_loop(1, n_c, chunk, (rm0, ra0, dot_chunk(0)))
    rm, ra = process(n_c - 1, d_last, rm, ra)

    wv = jnp.min(rm, axis=0, keepdims=True)
    cand = jnp.where(rm == wv, ra, jnp.int32(K))
    wi = jnp.min(cand, axis=0, keepdims=True)

    @pl.when(k == 0)
    def _():
        gv_ref[:, lane_sl] = wv
        gi_ref[:, lane_sl] = wi

    @pl.when(k > 0)
    def _():
        acc = gv_ref[:, lane_sl]
        av = acc.astype(jnp.bfloat16).astype(jnp.float32)
        upd = wv < av
        gv_ref[:, lane_sl] = jnp.where(upd, wv, av)
        gi_ref[:, lane_sl] = jnp.where(upd, wi, gi_ref[:, lane_sl])

    @pl.when(k == n_k - 1)
    def _():
        idx_ref[...] = gi_ref[:, lane_sl][None]


def _matmul_argmin(res, emb, e2, x2):
    n_b = B // B_TILE
    n_k = K // K_TILE
    idx3 = pl.pallas_call(
        functools.partial(_argmin_body, n_k=n_k),
        grid=(n_k, n_b),
        in_specs=[
            pl.BlockSpec((K_TILE, D), lambda k, b: (k, 0)),
            pl.BlockSpec((B_TILE, D), lambda k, b: (b, 0)),
            pl.BlockSpec((K_TILE, 1), lambda k, b: (k, 0)),
            pl.BlockSpec((1, B_TILE), lambda k, b: (0, b)),
        ],
        out_specs=pl.BlockSpec((1, 1, B_TILE), lambda k, b: (b, 0, 0)),
        out_shape=jax.ShapeDtypeStruct((n_b, 1, B_TILE), jnp.int32),
        scratch_shapes=[
            pltpu.VMEM((1, B), jnp.float32),
            pltpu.VMEM((1, B), jnp.int32),
        ],
    )(emb, res, e2, x2)
    return idx3.reshape(B)


def kernel(residual, codebooks):
    res = residual
    idxs = []
    for l in range(L):
        emb = codebooks[l]
        e2 = jnp.sum(emb ** 2, axis=1)[:, None]
        x2 = jnp.sum(res ** 2, axis=1)[None, :]
        idx = _matmul_argmin(res, emb, e2, x2)
        q = jnp.take(emb, idx, axis=0)
        res = res - q
        idxs.append(idx)
    codes = jnp.stack(idxs, axis=1)
    quantized_sum = residual - res
    return quantized_sum, codes


# speedup vs baseline: 2.3825x; 1.3650x over previous
import functools

import jax
import jax.numpy as jnp
from jax.experimental import pallas as pl
import jax.experimental.pallas.tpu as pltpu

B = 8192
D = 256
K = 8192
L = 4

B_TILE = 512
K_TILE = 4096
CHUNK = 64

BIG = 3.0e38


def _argmin_body(emb_ref, res_ref, e2_ref, x2_ref, idx_ref, gv_ref, gi_ref,
                 *, n_k):
    k = pl.program_id(0)
    b = pl.program_id(1)
    lane_sl = pl.ds(b * B_TILE, B_TILE)

    rs16 = res_ref[...].astype(jnp.bfloat16)
    x2r = x2_ref[...]
    sub_iota = jax.lax.broadcasted_iota(jnp.int32, (8, B_TILE), 0)
    base = k * K_TILE

    def dot_chunk(c):
        eblk16 = emb_ref[pl.ds(c * CHUNK, CHUNK), :].astype(jnp.bfloat16)
        return jax.lax.dot_general(
            eblk16, rs16, (((1,), (1,)), ((), ())),
            preferred_element_type=jnp.float32)

    def process(c, dots, rm, ra):
        e2c = e2_ref[pl.ds(c * CHUNK, CHUNK), :]
        scores = (x2r + e2c) - 2.0 * dots
        for g in range(CHUNK // 8):
            v = scores[g * 8:(g + 1) * 8, :]
            kidx = sub_iota + (base + c * CHUNK + g * 8)
            better = v < rm
            rm = jnp.minimum(rm, v)
            ra = jnp.where(better, kidx, ra)
        return rm, ra

    def chunk2(i, carry):
        rm, ra, d_prev = carry
        da = dot_chunk(2 * i)
        db = dot_chunk(2 * i + 1)
        rm, ra = process(2 * i - 1, d_prev, rm, ra)
        rm, ra = process(2 * i, da, rm, ra)
        return rm, ra, db

    n_c = K_TILE // CHUNK
    rm0 = jnp.full((8, B_TILE), BIG, jnp.float32)
    ra0 = jnp.zeros((8, B_TILE), jnp.int32)
    rm, ra = process(0, dot_chunk(0), rm0, ra0)
    rm, ra, d_last = jax.lax.fori_loop(
        1, n_c // 2, chunk2, (rm, ra, dot_chunk(1)))
    rm, ra = process(n_c - 1, d_last, rm, ra)

    wv = jnp.min(rm, axis=0, keepdims=True)
    cand = jnp.where(rm == wv, ra, jnp.int32(K))
    wi = jnp.min(cand, axis=0, keepdims=True)

    @pl.when(k == 0)
    def _():
        gv_ref[:, lane_sl] = wv
        gi_ref[:, lane_sl] = wi

    @pl.when(k > 0)
    def _():
        acc = gv_ref[:, lane_sl]
        av = acc.astype(jnp.bfloat16).astype(jnp.float32)
        upd = wv < av
        gv_ref[:, lane_sl] = jnp.where(upd, wv, av)
        gi_ref[:, lane_sl] = jnp.where(upd, wi, gi_ref[:, lane_sl])

    @pl.when(k == n_k - 1)
    def _():
        idx_ref[...] = gi_ref[:, lane_sl][None]


def _matmul_argmin(res, emb, e2, x2):
    n_b = B // B_TILE
    n_k = K // K_TILE
    idx3 = pl.pallas_call(
        functools.partial(_argmin_body, n_k=n_k),
        grid=(n_k, n_b),
        in_specs=[
            pl.BlockSpec((K_TILE, D), lambda k, b: (k, 0)),
            pl.BlockSpec((B_TILE, D), lambda k, b: (b, 0)),
            pl.BlockSpec((K_TILE, 1), lambda k, b: (k, 0)),
            pl.BlockSpec((1, B_TILE), lambda k, b: (0, b)),
        ],
        out_specs=pl.BlockSpec((1, 1, B_TILE), lambda k, b: (b, 0, 0)),
        out_shape=jax.ShapeDtypeStruct((n_b, 1, B_TILE), jnp.int32),
        scratch_shapes=[
            pltpu.VMEM((1, B), jnp.float32),
            pltpu.VMEM((1, B), jnp.int32),
        ],
    )(emb, res, e2, x2)
    return idx3.reshape(B)


def kernel(residual, codebooks):
    res = residual
    idxs = []
    for l in range(L):
        emb = codebooks[l]
        e2 = jnp.sum(emb ** 2, axis=1)[:, None]
        x2 = jnp.sum(res ** 2, axis=1)[None, :]
        idx = _matmul_argmin(res, emb, e2, x2)
        q = jnp.take(emb, idx, axis=0)
        res = res - q
        idxs.append(idx)
    codes = jnp.stack(idxs, axis=1)
    quantized_sum = residual - res
    return quantized_sum, codes
